# baseline (speedup 1.0000x reference)
import functools

import jax
import jax.numpy as jnp
from jax import lax
from jax.experimental import pallas as pl
from jax.experimental.pallas import tpu as pltpu
from jax.experimental.pallas import tpu_sc as plsc

_NC = 2
_NS = 16
_NW = _NC * _NS
_C = 256
_U = 8
_IR = _C // 128


def _tanh16(x):
    e = jnp.exp(-2.0 * jnp.abs(x))
    return jnp.sign(x) * (1.0 - e) / (1.0 + e)


@functools.lru_cache(maxsize=None)
def _make_sc_kernel(ncands, tot, nment, hp_d, ha_d):
    assert hp_d == 32 and ha_d == 32 and ncands % _NW == 0 and nment % 16 == 0
    bags_w = ncands // _NW
    mesh = plsc.VectorSubcoreMesh(core_axis_name="c", subcore_axis_name="s")

    @functools.partial(
        pl.kernel,
        out_type=jax.ShapeDtypeStruct((ncands,), jnp.float32),
        mesh=mesh,
        compiler_params=pltpu.CompilerParams(needs_layout_passes=False,
                                             use_tc_tiling_on_sc=False),
        scratch_types=[
            pltpu.VMEM((bags_w + 24, ), jnp.int32),
            pltpu.VMEM((bags_w,), jnp.int32),
            pltpu.VMEM((_IR, 128), jnp.int32),
            pltpu.VMEM((_C,), jnp.int32),
            pltpu.VMEM((_C, 128), jnp.float32),
            pltpu.VMEM((_C, 32), jnp.float32),
            pltpu.VMEM((32, nment), jnp.float32),
            pltpu.VMEM((nment,), jnp.float32),
            pltpu.VMEM((bags_w,), jnp.float32),
            pltpu.VMEM((32,), jnp.float32),
            pltpu.VMEM((72,), jnp.float32),
            pltpu.SemaphoreType.DMA,
        ],
    )
    def k(haT_hbm, phi_hbm, off_hbm, cand_hbm, table_hbm, bias_hbm, wb_hbm,
          out_hbm, off_v, cand_v, idx_v, sub_v, rows_v, pre_v, haT_v, t_v,
          sc_v, bias_v, wb_v, sem):
        wid = lax.axis_index("c") * _NS + lax.axis_index("s")
        base_bag = wid * bags_w

        pltpu.sync_copy(off_hbm.at[pl.ds(base_bag, bags_w + 24)], off_v)
        pltpu.sync_copy(cand_hbm.at[pl.ds(base_bag, bags_w)], cand_v)
        pltpu.sync_copy(bias_hbm, bias_v)
        pltpu.sync_copy(wb_hbm, wb_v)
        pltpu.sync_copy(haT_hbm, haT_v)

        bias_lo = bias_v[pl.ds(0, 16)]
        bias_hi = bias_v[pl.ds(16, 16)]
        w_lo = wb_v[pl.ds(0, 16)]
        w_hi = wb_v[pl.ds(16, 16)]
        wa_lo = wb_v[pl.ds(32, 16)]
        wa_hi = wb_v[pl.ds(48, 16)]
        b_sc = wb_v[pl.ds(56, 16)][8]
        lane = lax.iota(jnp.int32, 16)
        mask0 = lane == 0

        def sread(ref, i):
            return plsc.load_gather(ref, [jnp.full((16,), i, jnp.int32)])[0]

        def swrite(ref, i, x):
            plsc.store_scatter(ref, [jnp.full((16,), i, jnp.int32)],
                               jnp.full((16,), x, jnp.float32), mask=mask0)

        def t_body(g, _):
            acc = jnp.zeros((16,), jnp.float32)
            for c in range(32):
                wc = wa_lo[c] if c < 16 else wa_hi[c - 16]
                acc = acc + haT_v[c, pl.ds(g * 16, 16)] * wc
            t_v[pl.ds(g * 16, 16)] = acc + b_sc
            return 0

        lax.fori_loop(0, nment // 16, t_body, 0)

        def score_of(d_lo, d_hi):
            tl = _tanh16(d_lo + bias_lo)
            th = _tanh16(d_hi + bias_hi)
            return jnp.sum(tl * w_lo + th * w_hi)

        start = off_v[pl.ds(0, 16)][0]
        end = off_v[pl.ds(bags_w, 16)][0]
        astart = (start // 8) * 8
        shift = start - astart
        nchunks = (end - astart + _C - 1) // _C
        zv = jnp.zeros((16,), jnp.float32)
        s_empty = score_of(zv, zv)

        def pre_cond(cur):
            return (cur <= bags_w) & (sread(off_v, cur) <= start)

        def pre_body(cur):
            swrite(sc_v, cur - 1, s_empty)
            return cur + 1

        cur0 = lax.while_loop(pre_cond, pre_body, 1)

        def chunk_body(kc, carry):
            cur, lo, hi, pr_lo, pr_hi = carry
            cbase = astart + kc * _C

            for r in range(_IR):
                @pl.when(cbase + r * 128 < end)
                def _(r=r):
                    pltpu.sync_copy(phi_hbm.at[pl.ds(cbase + r * 128, 128)],
                                    idx_v.at[r])
            for r in range(_IR):
                for g in range(8):
                    v = idx_v[r, pl.ds(g * 16, 16)]
                    sub_v[pl.ds(r * 128 + g * 16, 16)] = (v & 3) << 5
                    idx_v[r, pl.ds(g * 16, 16)] = lax.shift_right_logical(v, 2)
            for r in range(_IR):
                @pl.when(cbase + r * 128 < end)
                def _(r=r):
                    pltpu.async_copy(table_hbm.at[idx_v.at[r]],
                                     rows_v.at[pl.ds(r * 128, 128)],
                                     sem).wait()

            @pl.when(kc == 0)
            def _():
                for u in range(8):
                    @pl.when(u < shift)
                    def _(u=u):
                        su = sread(sub_v, u)
                        rows_v[u, pl.ds(su, 16)] = zv
                        rows_v[u, pl.ds(su + 16, 16)] = zv

            def p_body(i, c2):
                lo2, hi2 = c2
                j0 = i * _U
                for u in range(_U):
                    su = sread(sub_v, j0 + u)
                    lo2 = lo2 + rows_v[j0 + u, pl.ds(su, 16)]
                    hi2 = hi2 + rows_v[j0 + u, pl.ds(su + 16, 16)]
                    pre_v[j0 + u, pl.ds(0, 16)] = lo2
                    pre_v[j0 + u, pl.ds(16, 16)] = hi2
                return (lo2, hi2)

            lo, hi = lax.fori_loop(0, _C // _U, p_body, (lo, hi))

            lim = cbase + _C

            def b_cond(c3):
                return (c3[0] <= bags_w) & (sread(off_v, c3[0]) <= lim)

            def b_body(c3):
                cur3, q_lo, q_hi = c3
                p = sread(off_v, cur3)
                pv_lo = pre_v[p - 1 - cbase, pl.ds(0, 16)]
                pv_hi = pre_v[p - 1 - cbase, pl.ds(16, 16)]
                swrite(sc_v, cur3 - 1, score_of(pv_lo - q_lo, pv_hi - q_hi))
                return (cur3 + 1, pv_lo, pv_hi)

            cur, pr_lo, pr_hi = lax.while_loop(b_cond, b_body,
                                               (cur, pr_lo, pr_hi))
            return (cur, lo, hi, pr_lo, pr_hi)

        lax.fori_loop(0, nchunks, chunk_body, (cur0, zv, zv, zv, zv))

        def c_body(g, _):
            cand_g = cand_v[pl.ds(g * 16, 16)]
            tg = plsc.load_gather(t_v, [cand_g])
            sc_v[pl.ds(g * 16, 16)] = sc_v[pl.ds(g * 16, 16)] + tg
            return 0

        lax.fori_loop(0, bags_w // 16, c_body, 0)
        pltpu.sync_copy(sc_v, out_hbm.at[pl.ds(base_bag, bags_w)])

    return k


def kernel(h_a, phi_p, phi_p_offsets, cand_subset, emb_table, hp_bias, W, b):
    ncands = cand_subset.shape[0]
    tot = phi_p.shape[0]
    nment, ha_d = h_a.shape
    hp_d = emb_table.shape[1]
    nfeat = emb_table.shape[0]

    haT = jnp.asarray(h_a.T, dtype=jnp.float32)
    pad_idx = (jnp.arange(_C, dtype=jnp.int32) * 64) % nfeat
    phi_pad = jnp.concatenate([phi_p.astype(jnp.int32), pad_idx])
    off_pad = jnp.concatenate(
        [phi_p_offsets.astype(jnp.int32), jnp.zeros((23,), jnp.int32)])
    wb = jnp.concatenate(
        [W[0].astype(jnp.float32), b.astype(jnp.float32),
         jnp.zeros((7,), jnp.float32)])

    table4 = emb_table.reshape(-1, 128)

    fn = _make_sc_kernel(ncands, tot, nment, hp_d, ha_d)
    return fn(haT, phi_pad, off_pad, cand_subset.astype(jnp.int32), table4,
              hp_bias.astype(jnp.float32), wb)


# speedup vs baseline: 1.0357x; 1.0357x over previous
import functools

import jax
import jax.numpy as jnp
from jax import lax
from jax.experimental import pallas as pl
from jax.experimental.pallas import tpu as pltpu
from jax.experimental.pallas import tpu_sc as plsc

_NC = 2
_NS = 16
_NW = _NC * _NS
_C = 256
_U = 8
_IR = _C // 128


def _tanh16(x):
    e = jnp.exp(-2.0 * jnp.abs(x))
    return jnp.sign(x) * (1.0 - e) / (1.0 + e)


@functools.lru_cache(maxsize=None)
def _make_sc_kernel(ncands, tot, nment, hp_d, ha_d):
    assert hp_d == 32 and ha_d == 32 and ncands % _NW == 0 and nment % 128 == 0
    bags_w = ncands // _NW
    mesh = plsc.VectorSubcoreMesh(core_axis_name="c", subcore_axis_name="s")

    @functools.partial(
        pl.kernel,
        out_type=jax.ShapeDtypeStruct((ncands,), jnp.float32),
        mesh=mesh,
        compiler_params=pltpu.CompilerParams(needs_layout_passes=False,
                                             use_tc_tiling_on_sc=True),
        scratch_types=[
            pltpu.VMEM((bags_w + 24,), jnp.int32),
            pltpu.VMEM((bags_w,), jnp.int32),
            pltpu.VMEM((_IR, 128), jnp.int32),
            pltpu.VMEM((_C,), jnp.int32),
            pltpu.VMEM((_C, 128), jnp.float32),
            pltpu.VMEM((_C, 32), jnp.float32),
            pltpu.VMEM((32, 128), jnp.float32),
            pltpu.VMEM((nment,), jnp.float32),
            pltpu.VMEM((bags_w,), jnp.float32),
            pltpu.VMEM((32,), jnp.float32),
            pltpu.VMEM((72,), jnp.float32),
            pltpu.SemaphoreType.DMA,
        ],
    )
    def k(haT_hbm, phi_hbm, off_hbm, cand_hbm, table_hbm, bias_hbm, wb_hbm,
          out_hbm, off_v, cand_v, idx_v, sub_v, rows_v, pre_v, hab_v,
          t_v, sc_v, bias_v, wb_v, sem):
        wid = lax.axis_index("c") * _NS + lax.axis_index("s")
        base_bag = wid * bags_w

        pltpu.sync_copy(off_hbm.at[pl.ds(base_bag, bags_w + 24)], off_v)
        pltpu.sync_copy(cand_hbm.at[pl.ds(base_bag, bags_w)], cand_v)
        pltpu.sync_copy(bias_hbm, bias_v)
        pltpu.sync_copy(wb_hbm, wb_v)

        bias_lo = bias_v[pl.ds(0, 16)]
        bias_hi = bias_v[pl.ds(16, 16)]
        w_lo = wb_v[pl.ds(0, 16)]
        w_hi = wb_v[pl.ds(16, 16)]
        wa_lo = wb_v[pl.ds(32, 16)]
        wa_hi = wb_v[pl.ds(48, 16)]
        b_sc = wb_v[pl.ds(56, 16)][8]
        lane = lax.iota(jnp.int32, 16)
        lane4 = lane * 4
        mask0 = lane == 0

        def sread(ref, i):
            return plsc.load_gather(ref, [jnp.full((16,), i, jnp.int32)])[0]

        def swrite(ref, i, x):
            plsc.store_scatter(ref, [jnp.full((16,), i, jnp.int32)],
                               jnp.full((16,), x, jnp.float32), mask=mask0)

        for blk in range(nment // 128):
            pltpu.sync_copy(haT_hbm.at[:, pl.ds(blk * 128, 128)], hab_v)

            def t_body(g, _, blk=blk):
                acc = jnp.zeros((16,), jnp.float32)
                for c in range(32):
                    wc = wa_lo[c] if c < 16 else wa_hi[c - 16]
                    acc = acc + hab_v[c, pl.ds(g * 16, 16)] * wc
                t_v[pl.ds(blk * 128 + g * 16, 16)] = acc + b_sc
                return 0

            lax.fori_loop(0, 8, t_body, 0)

        def score_of(d_lo, d_hi):
            tl = _tanh16(d_lo + bias_lo)
            th = _tanh16(d_hi + bias_hi)
            return jnp.sum(tl * w_lo + th * w_hi)

        start = off_v[pl.ds(0, 16)][0]
        end = off_v[pl.ds(bags_w, 16)][0]
        astart = (start // 128) * 128
        shift = start - astart
        nchunks = (end - astart + _C - 1) // _C
        zv = jnp.zeros((16,), jnp.float32)
        s_empty = score_of(zv, zv)

        def pre_cond(cur):
            return (cur <= bags_w) & (sread(off_v, cur) <= start)

        def pre_body(cur):
            swrite(sc_v, cur - 1, s_empty)
            return cur + 1

        cur0 = lax.while_loop(pre_cond, pre_body, 1)

        def chunk_body(kc, carry):
            cur, lo, hi, pr_lo, pr_hi = carry
            cbase = astart + kc * _C

            cps = [pltpu.async_copy(phi_hbm.at[pl.ds(cbase + r * 128, 128)],
                                    idx_v.at[r], sem) for r in range(_IR)]
            for cp in cps:
                cp.wait()
            for r in range(_IR):
                for g in range(8):
                    j = r * 128 + g * 16
                    v = idx_v[r, pl.ds(g * 16, 16)]
                    sub_v[pl.ds(j, 16)] = (v & 3) << 5
                    idx_v[r, pl.ds(g * 16, 16)] = lax.shift_right_logical(v, 2)
            cps = [pltpu.async_copy(table_hbm.at[idx_v.at[r]],
                                    rows_v.at[pl.ds(r * 128, 128)], sem)
                   for r in range(_IR)]
            for cp in cps:
                cp.wait()

            def p_body(i, c2):
                lo2, hi2 = c2
                j0 = i * _U
                for u in range(_U):
                    su = sread(sub_v, j0 + u)
                    lo2 = lo2 + rows_v[j0 + u, pl.ds(su, 16)]
                    hi2 = hi2 + rows_v[j0 + u, pl.ds(su + 16, 16)]
                    pre_v[j0 + u, pl.ds(0, 16)] = lo2
                    pre_v[j0 + u, pl.ds(16, 16)] = hi2
                return (lo2, hi2)

            lo, hi = lax.fori_loop(0, _C // _U, p_body, (lo, hi))

            sidx = jnp.maximum(shift - 1, 0)
            use = (kc == 0) & (shift > 0)
            pr_lo = jnp.where(use, pre_v[sidx, pl.ds(0, 16)], pr_lo)
            pr_hi = jnp.where(use, pre_v[sidx, pl.ds(16, 16)], pr_hi)

            lim = cbase + _C

            def b_cond(c3):
                return (c3[0] <= bags_w) & (sread(off_v, c3[0]) <= lim)

            def b_body(c3):
                cur3, q_lo, q_hi = c3
                p = sread(off_v, cur3)
                pv_lo = pre_v[p - 1 - cbase, pl.ds(0, 16)]
                pv_hi = pre_v[p - 1 - cbase, pl.ds(16, 16)]
                swrite(sc_v, cur3 - 1, score_of(pv_lo - q_lo, pv_hi - q_hi))
                return (cur3 + 1, pv_lo, pv_hi)

            cur, pr_lo, pr_hi = lax.while_loop(b_cond, b_body,
                                               (cur, pr_lo, pr_hi))
            return (cur, lo, hi, pr_lo, pr_hi)

        lax.fori_loop(0, nchunks, chunk_body, (cur0, zv, zv, zv, zv))

        def c_body(g, _):
            cand_g = cand_v[pl.ds(g * 16, 16)]
            tg = plsc.load_gather(t_v, [cand_g])
            sc_v[pl.ds(g * 16, 16)] = sc_v[pl.ds(g * 16, 16)] + tg
            return 0

        lax.fori_loop(0, bags_w // 16, c_body, 0)
        pltpu.sync_copy(sc_v, out_hbm.at[pl.ds(base_bag, bags_w)])

    return k


def kernel(h_a, phi_p, phi_p_offsets, cand_subset, emb_table, hp_bias, W, b):
    ncands = cand_subset.shape[0]
    tot = phi_p.shape[0]
    nment, ha_d = h_a.shape
    hp_d = emb_table.shape[1]
    nfeat = emb_table.shape[0]

    haT = jnp.asarray(h_a.T, dtype=jnp.float32)
    pad_idx = (jnp.arange(_C, dtype=jnp.int32) * 64) % nfeat
    phi_pad = jnp.concatenate([phi_p.astype(jnp.int32), pad_idx])
    off_pad = jnp.concatenate(
        [phi_p_offsets.astype(jnp.int32), jnp.zeros((23,), jnp.int32)])
    wb = jnp.concatenate(
        [W[0].astype(jnp.float32), b.astype(jnp.float32),
         jnp.zeros((7,), jnp.float32)])

    table4 = emb_table.reshape(-1, 128)

    fn = _make_sc_kernel(ncands, tot, nment, hp_d, ha_d)
    return fn(haT, phi_pad, off_pad, cand_subset.astype(jnp.int32), table4,
              hp_bias.astype(jnp.float32), wb)


# speedup vs baseline: 1.5706x; 1.5164x over previous
import functools

import jax
import jax.numpy as jnp
from jax import lax
from jax.experimental import pallas as pl
from jax.experimental.pallas import tpu as pltpu
from jax.experimental.pallas import tpu_sc as plsc

_NC = 2
_NS = 16
_NW = _NC * _NS
_C = 1024
_U = 8
_IR = _C // 128


def _tanh16(x):
    e = jnp.exp(-2.0 * jnp.abs(x))
    return jnp.sign(x) * (1.0 - e) / (1.0 + e)


@functools.lru_cache(maxsize=None)
def _make_sc_kernel(ncands, tot, nment, hp_d, ha_d, nfeat):
    assert hp_d == 32 and ha_d == 32 and ncands % _NW == 0 and nment % 128 == 0
    bags_w = ncands // _NW
    mesh = plsc.VectorSubcoreMesh(core_axis_name="c", subcore_axis_name="s")

    @functools.partial(
        pl.kernel,
        out_type=jax.ShapeDtypeStruct((ncands,), jnp.float32),
        mesh=mesh,
        compiler_params=pltpu.CompilerParams(needs_layout_passes=False,
                                             use_tc_tiling_on_sc=False),
        scratch_types=[
            pltpu.VMEM((bags_w + 24,), jnp.int32),
            pltpu.VMEM((bags_w,), jnp.int32),
            pltpu.VMEM((_IR, 128), jnp.int32),
            pltpu.VMEM((_C, 32), jnp.float32),
            pltpu.VMEM((_C, 32), jnp.float32),
            pltpu.VMEM((32, 128), jnp.float32),
            pltpu.VMEM((nment,), jnp.float32),
            pltpu.VMEM((bags_w,), jnp.float32),
            pltpu.VMEM((32,), jnp.float32),
            pltpu.VMEM((72,), jnp.float32),
            pltpu.SemaphoreType.DMA,
        ],
    )
    def k(haT_hbm, phi_hbm, off_hbm, cand_hbm, table_hbm, bias_hbm, wb_hbm,
          out_hbm, off_v, cand_v, idx_v, rows_v, pre_v, hab_v,
          t_v, sc_v, bias_v, wb_v, sem):
        tbl = table_hbm
        wid = lax.axis_index("c") * _NS + lax.axis_index("s")
        base_bag = wid * bags_w

        pltpu.sync_copy(off_hbm.at[pl.ds(base_bag, bags_w + 24)], off_v)
        pltpu.sync_copy(cand_hbm.at[pl.ds(base_bag, bags_w)], cand_v)
        pltpu.sync_copy(bias_hbm, bias_v)
        pltpu.sync_copy(wb_hbm, wb_v)

        bias_lo = bias_v[pl.ds(0, 16)]
        bias_hi = bias_v[pl.ds(16, 16)]
        w_lo = wb_v[pl.ds(0, 16)]
        w_hi = wb_v[pl.ds(16, 16)]
        wa_lo = wb_v[pl.ds(32, 16)]
        wa_hi = wb_v[pl.ds(48, 16)]
        b_sc = wb_v[pl.ds(56, 16)][8]
        lane = lax.iota(jnp.int32, 16)
        lane4 = lane * 4
        mask0 = lane == 0

        def sread(ref, i):
            return plsc.load_gather(ref, [jnp.full((16,), i, jnp.int32)])[0]

        def swrite(ref, i, x):
            plsc.store_scatter(ref, [jnp.full((16,), i, jnp.int32)],
                               jnp.full((16,), x, jnp.float32), mask=mask0)

        for blk in range(nment // 128):
            pltpu.sync_copy(haT_hbm.at[:, pl.ds(blk * 128, 128)], hab_v)

            def t_body(g, _, blk=blk):
                acc = jnp.zeros((16,), jnp.float32)
                for c in range(32):
                    wc = wa_lo[c] if c < 16 else wa_hi[c - 16]
                    acc = acc + hab_v[c, pl.ds(g * 16, 16)] * wc
                t_v[pl.ds(blk * 128 + g * 16, 16)] = acc + b_sc
                return 0

            lax.fori_loop(0, 8, t_body, 0)

        def score_of(d_lo, d_hi):
            tl = _tanh16(d_lo + bias_lo)
            th = _tanh16(d_hi + bias_hi)
            return jnp.sum(tl * w_lo + th * w_hi)

        start = off_v[pl.ds(0, 16)][0]
        end = off_v[pl.ds(bags_w, 16)][0]
        astart = (start // 128) * 128
        shift = start - astart
        nchunks = (end - astart + _C - 1) // _C
        zv = jnp.zeros((16,), jnp.float32)
        s_empty = score_of(zv, zv)

        def pre_cond(cur):
            return (cur <= bags_w) & (sread(off_v, cur) <= start)

        def pre_body(cur):
            swrite(sc_v, cur - 1, s_empty)
            return cur + 1

        cur0 = lax.while_loop(pre_cond, pre_body, 1)

        def chunk_body(kc, carry):
            cur, lo, hi, pr_lo, pr_hi = carry
            cbase = astart + kc * _C

            cps = [pltpu.async_copy(phi_hbm.at[pl.ds(cbase + r * 128, 128)],
                                    idx_v.at[r], sem) for r in range(_IR)]
            for cp in cps:
                cp.wait()
            cps = [pltpu.async_copy(tbl.at[idx_v.at[r]],
                                    rows_v.at[pl.ds(r * 128, 128)], sem)
                   for r in range(_IR)]
            for cp in cps:
                cp.wait()

            def p_body(i, c2):
                lo2, hi2 = c2
                j0 = i * _U
                for u in range(_U):
                    lo2 = lo2 + rows_v[j0 + u, pl.ds(0, 16)]
                    hi2 = hi2 + rows_v[j0 + u, pl.ds(16, 16)]
                    pre_v[j0 + u, pl.ds(0, 16)] = lo2
                    pre_v[j0 + u, pl.ds(16, 16)] = hi2
                return (lo2, hi2)

            lo, hi = lax.fori_loop(0, _C // _U, p_body, (lo, hi))

            sidx = jnp.maximum(shift - 1, 0)
            use = (kc == 0) & (shift > 0)
            pr_lo = jnp.where(use, pre_v[sidx, pl.ds(0, 16)], pr_lo)
            pr_hi = jnp.where(use, pre_v[sidx, pl.ds(16, 16)], pr_hi)

            lim = cbase + _C

            def b_cond(c3):
                return (c3[0] <= bags_w) & (sread(off_v, c3[0]) <= lim)

            def b_body(c3):
                cur3, q_lo, q_hi = c3
                p = sread(off_v, cur3)
                pv_lo = pre_v[p - 1 - cbase, pl.ds(0, 16)]
                pv_hi = pre_v[p - 1 - cbase, pl.ds(16, 16)]
                swrite(sc_v, cur3 - 1, score_of(pv_lo - q_lo, pv_hi - q_hi))
                return (cur3 + 1, pv_lo, pv_hi)

            cur, pr_lo, pr_hi = lax.while_loop(b_cond, b_body,
                                               (cur, pr_lo, pr_hi))
            return (cur, lo, hi, pr_lo, pr_hi)

        lax.fori_loop(0, nchunks, chunk_body, (cur0, zv, zv, zv, zv))

        def c_body(g, _):
            cand_g = cand_v[pl.ds(g * 16, 16)]
            tg = plsc.load_gather(t_v, [cand_g])
            sc_v[pl.ds(g * 16, 16)] = sc_v[pl.ds(g * 16, 16)] + tg
            return 0

        lax.fori_loop(0, bags_w // 16, c_body, 0)
        pltpu.sync_copy(sc_v, out_hbm.at[pl.ds(base_bag, bags_w)])

    return k


def kernel(h_a, phi_p, phi_p_offsets, cand_subset, emb_table, hp_bias, W, b):
    ncands = cand_subset.shape[0]
    tot = phi_p.shape[0]
    nment, ha_d = h_a.shape
    hp_d = emb_table.shape[1]
    nfeat = emb_table.shape[0]

    haT = jnp.asarray(h_a.T, dtype=jnp.float32)
    pad_idx = (jnp.arange(_C, dtype=jnp.int32) * 64) % nfeat
    phi_pad = jnp.concatenate([phi_p.astype(jnp.int32), pad_idx])
    off_pad = jnp.concatenate(
        [phi_p_offsets.astype(jnp.int32), jnp.zeros((23,), jnp.int32)])
    wb = jnp.concatenate(
        [W[0].astype(jnp.float32), b.astype(jnp.float32),
         jnp.zeros((7,), jnp.float32)])

    fn = _make_sc_kernel(ncands, tot, nment, hp_d, ha_d, nfeat)
    return fn(haT, phi_pad, off_pad, cand_subset.astype(jnp.int32), emb_table,
              hp_bias.astype(jnp.float32), wb)


# speedup vs baseline: 1.5755x; 1.0031x over previous
import functools

import jax
import jax.numpy as jnp
from jax import lax
from jax.experimental import pallas as pl
from jax.experimental.pallas import tpu as pltpu
from jax.experimental.pallas import tpu_sc as plsc

_NC = 2
_NS = 16
_NW = _NC * _NS
_C = 1024
_U = 8
_IR = _C // 128


def _tanh16(x):
    e = jnp.exp(-2.0 * jnp.abs(x))
    return jnp.sign(x) * (1.0 - e) / (1.0 + e)


@functools.lru_cache(maxsize=None)
def _make_sc_kernel(ncands, tot, nment, hp_d, ha_d, nfeat):
    assert hp_d == 32 and ha_d == 32 and ncands % _NW == 0 and nment % 128 == 0
    bags_w = ncands // _NW
    mesh = plsc.VectorSubcoreMesh(core_axis_name="c", subcore_axis_name="s")

    @functools.partial(
        pl.kernel,
        out_type=jax.ShapeDtypeStruct((ncands,), jnp.float32),
        mesh=mesh,
        compiler_params=pltpu.CompilerParams(needs_layout_passes=False,
                                             use_tc_tiling_on_sc=False),
        scratch_types=[
            pltpu.VMEM((bags_w + 24,), jnp.int32),
            pltpu.VMEM((bags_w,), jnp.int32),
            pltpu.VMEM((_IR, 128), jnp.int32),
            pltpu.VMEM((_C, 32), jnp.float32),
            pltpu.VMEM((_C, 32), jnp.float32),
            pltpu.VMEM((32, 128), jnp.float32),
            pltpu.VMEM((nment,), jnp.float32),
            pltpu.VMEM((bags_w,), jnp.float32),
            pltpu.VMEM((32,), jnp.float32),
            pltpu.VMEM((72,), jnp.float32),
            pltpu.SemaphoreType.DMA,
        ],
    )
    def k(haT_hbm, phi_hbm, off_hbm, cand_hbm, table_hbm, bias_hbm, wb_hbm,
          out_hbm, off_v, cand_v, idx_v, rows_v, pre_v, hab_v,
          t_v, sc_v, bias_v, wb_v, sem):
        tbl = table_hbm
        wid = lax.axis_index("c") * _NS + lax.axis_index("s")
        base_bag = wid * bags_w

        pltpu.sync_copy(off_hbm.at[pl.ds(base_bag, bags_w + 24)], off_v)
        pltpu.sync_copy(cand_hbm.at[pl.ds(base_bag, bags_w)], cand_v)
        pltpu.sync_copy(bias_hbm, bias_v)
        pltpu.sync_copy(wb_hbm, wb_v)

        bias_lo = bias_v[pl.ds(0, 16)]
        bias_hi = bias_v[pl.ds(16, 16)]
        w_lo = wb_v[pl.ds(0, 16)]
        w_hi = wb_v[pl.ds(16, 16)]
        wa_lo = wb_v[pl.ds(32, 16)]
        wa_hi = wb_v[pl.ds(48, 16)]
        b_sc = wb_v[pl.ds(56, 16)][8]
        lane = lax.iota(jnp.int32, 16)
        lane4 = lane * 4
        mask0 = lane == 0

        def sread(ref, i):
            return ref[pl.ds(i, 16)][0]

        def swrite(ref, i, x):
            plsc.store_scatter(ref, [jnp.full((16,), i, jnp.int32)],
                               jnp.full((16,), x, jnp.float32), mask=mask0)

        for blk in range(nment // 128):
            pltpu.sync_copy(haT_hbm.at[:, pl.ds(blk * 128, 128)], hab_v)

            def t_body(g, _, blk=blk):
                acc = jnp.zeros((16,), jnp.float32)
                for c in range(32):
                    wc = wa_lo[c] if c < 16 else wa_hi[c - 16]
                    acc = acc + hab_v[c, pl.ds(g * 16, 16)] * wc
                t_v[pl.ds(blk * 128 + g * 16, 16)] = acc + b_sc
                return 0

            lax.fori_loop(0, 8, t_body, 0)

        def score_of(d_lo, d_hi):
            tl = _tanh16(d_lo + bias_lo)
            th = _tanh16(d_hi + bias_hi)
            return jnp.sum(tl * w_lo + th * w_hi)

        start = off_v[pl.ds(0, 16)][0]
        end = off_v[pl.ds(bags_w, 16)][0]
        astart = (start // 128) * 128
        shift = start - astart
        nchunks = (end - astart + _C - 1) // _C
        zv = jnp.zeros((16,), jnp.float32)
        s_empty = score_of(zv, zv)

        def pre_cond(cur):
            return (cur <= bags_w) & (sread(off_v, cur) <= start)

        def pre_body(cur):
            swrite(sc_v, cur - 1, s_empty)
            return cur + 1

        cur0 = lax.while_loop(pre_cond, pre_body, 1)

        def chunk_body(kc, carry):
            cur, lo, hi, pr_lo, pr_hi = carry
            cbase = astart + kc * _C

            cps = [pltpu.async_copy(phi_hbm.at[pl.ds(cbase + r * 128, 128)],
                                    idx_v.at[r], sem) for r in range(_IR)]
            for cp in cps:
                cp.wait()
            cps = [pltpu.async_copy(tbl.at[idx_v.at[r]],
                                    rows_v.at[pl.ds(r * 128, 128)], sem)
                   for r in range(_IR)]
            for cp in cps:
                cp.wait()

            def p_body(i, c2):
                lo2, hi2 = c2
                j0 = i * _U
                for u in range(_U):
                    lo2 = lo2 + rows_v[j0 + u, pl.ds(0, 16)]
                    hi2 = hi2 + rows_v[j0 + u, pl.ds(16, 16)]
                    pre_v[j0 + u, pl.ds(0, 16)] = lo2
                    pre_v[j0 + u, pl.ds(16, 16)] = hi2
                return (lo2, hi2)

            lo, hi = lax.fori_loop(0, _C // _U, p_body, (lo, hi))

            sidx = jnp.maximum(shift - 1, 0)
            use = (kc == 0) & (shift > 0)
            pr_lo = jnp.where(use, pre_v[sidx, pl.ds(0, 16)], pr_lo)
            pr_hi = jnp.where(use, pre_v[sidx, pl.ds(16, 16)], pr_hi)

            lim = cbase + _C

            def b_cond(c3):
                return (c3[0] <= bags_w) & (sread(off_v, c3[0]) <= lim)

            def b_body(c3):
                cur3, q_lo, q_hi = c3
                p = sread(off_v, cur3)
                pv_lo = pre_v[p - 1 - cbase, pl.ds(0, 16)]
                pv_hi = pre_v[p - 1 - cbase, pl.ds(16, 16)]
                swrite(sc_v, cur3 - 1, score_of(pv_lo - q_lo, pv_hi - q_hi))
                return (cur3 + 1, pv_lo, pv_hi)

            cur, pr_lo, pr_hi = lax.while_loop(b_cond, b_body,
                                               (cur, pr_lo, pr_hi))
            return (cur, lo, hi, pr_lo, pr_hi)

        lax.fori_loop(0, nchunks, chunk_body, (cur0, zv, zv, zv, zv))

        def c_body(g, _):
            cand_g = cand_v[pl.ds(g * 16, 16)]
            tg = plsc.load_gather(t_v, [cand_g])
            sc_v[pl.ds(g * 16, 16)] = sc_v[pl.ds(g * 16, 16)] + tg
            return 0

        lax.fori_loop(0, bags_w // 16, c_body, 0)
        pltpu.sync_copy(sc_v, out_hbm.at[pl.ds(base_bag, bags_w)])

    return k


def kernel(h_a, phi_p, phi_p_offsets, cand_subset, emb_table, hp_bias, W, b):
    ncands = cand_subset.shape[0]
    tot = phi_p.shape[0]
    nment, ha_d = h_a.shape
    hp_d = emb_table.shape[1]
    nfeat = emb_table.shape[0]

    haT = jnp.asarray(h_a.T, dtype=jnp.float32)
    pad_idx = (jnp.arange(_C, dtype=jnp.int32) * 64) % nfeat
    phi_pad = jnp.concatenate([phi_p.astype(jnp.int32), pad_idx])
    off_pad = jnp.concatenate(
        [phi_p_offsets.astype(jnp.int32), jnp.zeros((23,), jnp.int32)])
    wb = jnp.concatenate(
        [W[0].astype(jnp.float32), b.astype(jnp.float32),
         jnp.zeros((7,), jnp.float32)])

    fn = _make_sc_kernel(ncands, tot, nment, hp_d, ha_d, nfeat)
    return fn(haT, phi_pad, off_pad, cand_subset.astype(jnp.int32), emb_table,
              hp_bias.astype(jnp.float32), wb)
